# baseline (speedup 1.0000x reference)
import jax
import jax.numpy as jnp
from jax.experimental import pallas as pl
from jax.experimental.pallas import tpu as pltpu

_NSUB = 4


def _scan_kernel(x_ref, b_ref, g1_ref, g2_ref, g3_ref, o_ref, *scratch):
    bcb, H, W = x_ref.shape
    sub = bcb // _NSUB
    zero = jnp.zeros((sub, 1), jnp.float32)

    for k in range(_NSUB):
        bxs, g1s, g2s, g3s, os = scratch[5 * k:5 * k + 5]
        sl = slice(k * sub, (k + 1) * sub)

        bxs[...] = jnp.transpose(x_ref[sl] * b_ref[sl], (2, 0, 1))
        g1s[...] = jnp.transpose(g1_ref[sl], (2, 0, 1))
        g2s[...] = jnp.transpose(g2_ref[sl], (2, 0, 1))
        g3s[...] = jnp.transpose(g3_ref[sl], (2, 0, 1))

        h = jnp.zeros((sub, H), jnp.float32)
        for w in range(W):
            up = jnp.concatenate([zero, h[:, :-1]], axis=1)
            dn = jnp.concatenate([h[:, 1:], zero], axis=1)
            h = bxs[w] + g1s[w] * up + g2s[w] * h + g3s[w] * dn
            os[w] = h

        o_ref[sl] = jnp.transpose(os[...], (1, 2, 0))


def kernel(X, B, G1, G2, G3):
    Bsz, C, H, W = X.shape
    BC = Bsz * C
    bcb = min(32, BC)
    sub = bcb // _NSUB

    ins = [t.reshape(BC, H, W) for t in (X, B, G1, G2, G3)]

    spec = pl.BlockSpec((bcb, H, W), lambda i: (i, 0, 0))
    scratch = []
    for _ in range(_NSUB):
        scratch += [pltpu.VMEM((W, sub, H), jnp.float32) for _ in range(5)]
    out = pl.pallas_call(
        _scan_kernel,
        grid=(BC // bcb,),
        in_specs=[spec] * 5,
        out_specs=spec,
        out_shape=jax.ShapeDtypeStruct((BC, H, W), jnp.float32),
        scratch_shapes=scratch,
        compiler_params=pltpu.CompilerParams(
            dimension_semantics=("parallel",),
            vmem_limit_bytes=100 * 1024 * 1024,
        ),
    )(*ins)
    return out.reshape(Bsz, C, H, W)


# speedup vs baseline: 2.0659x; 2.0659x over previous
import jax
import jax.numpy as jnp
from jax.experimental import pallas as pl
from jax.experimental.pallas import tpu as pltpu

_NSUB = 2


def _scan_kernel(x_ref, b_ref, g1_ref, g2_ref, g3_ref, o_ref, *scratch):
    bcb, H, W = x_ref.shape
    sub = bcb // _NSUB
    zero = jnp.zeros((sub, 1), jnp.float32)

    for k in range(_NSUB):
        bxs, g1s, g2s, g3s, os = scratch[5 * k:5 * k + 5]
        sl = slice(k * sub, (k + 1) * sub)

        bxs[...] = jnp.transpose(x_ref[sl] * b_ref[sl], (2, 0, 1))
        g1s[...] = jnp.transpose(g1_ref[sl], (2, 0, 1))
        g2s[...] = jnp.transpose(g2_ref[sl], (2, 0, 1))
        g3s[...] = jnp.transpose(g3_ref[sl], (2, 0, 1))

        h = jnp.zeros((sub, H), jnp.float32)
        for w in range(W):
            up = jnp.concatenate([zero, h[:, :-1]], axis=1)
            dn = jnp.concatenate([h[:, 1:], zero], axis=1)
            h = bxs[w] + g1s[w] * up + g2s[w] * h + g3s[w] * dn
            os[w] = h

        o_ref[sl] = jnp.transpose(os[...], (1, 2, 0))


def kernel(X, B, G1, G2, G3):
    Bsz, C, H, W = X.shape
    BC = Bsz * C
    bcb = min(32, BC)
    sub = bcb // _NSUB

    ins = [t.reshape(BC, H, W) for t in (X, B, G1, G2, G3)]

    spec = pl.BlockSpec((bcb, H, W), lambda i: (i, 0, 0))
    scratch = []
    for _ in range(_NSUB):
        scratch += [pltpu.VMEM((W, sub, H), jnp.float32) for _ in range(5)]
    out = pl.pallas_call(
        _scan_kernel,
        grid=(BC // bcb,),
        in_specs=[spec] * 5,
        out_specs=spec,
        out_shape=jax.ShapeDtypeStruct((BC, H, W), jnp.float32),
        scratch_shapes=scratch,
        compiler_params=pltpu.CompilerParams(
            dimension_semantics=("parallel",),
            vmem_limit_bytes=100 * 1024 * 1024,
        ),
    )(*ins)
    return out.reshape(Bsz, C, H, W)


# speedup vs baseline: 3.1770x; 1.5379x over previous
import jax
import jax.numpy as jnp
from jax.experimental import pallas as pl
from jax.experimental.pallas import tpu as pltpu


def _scan_kernel(x_ref, b_ref, g1_ref, g2_ref, g3_ref, o_ref,
                 bxs, g1s, g2s, g3s, os):
    bcb, H, W = x_ref.shape

    bxs[...] = jnp.transpose(x_ref[...] * b_ref[...], (2, 0, 1))
    g1s[...] = jnp.transpose(g1_ref[...], (2, 0, 1))
    g2s[...] = jnp.transpose(g2_ref[...], (2, 0, 1))
    g3s[...] = jnp.transpose(g3_ref[...], (2, 0, 1))

    zero = jnp.zeros((bcb, 1), jnp.float32)
    h = jnp.zeros((bcb, H), jnp.float32)
    for w in range(W):
        up = jnp.concatenate([zero, h[:, :-1]], axis=1)
        dn = jnp.concatenate([h[:, 1:], zero], axis=1)
        h = bxs[w] + g1s[w] * up + g2s[w] * h + g3s[w] * dn
        os[w] = h

    o_ref[...] = jnp.transpose(os[...], (1, 2, 0))


def kernel(X, B, G1, G2, G3):
    Bsz, C, H, W = X.shape
    BC = Bsz * C
    bcb = min(32, BC)

    ins = [t.reshape(BC, H, W) for t in (X, B, G1, G2, G3)]

    spec = pl.BlockSpec((bcb, H, W), lambda i: (i, 0, 0))
    scratch = [pltpu.VMEM((W, bcb, H), jnp.float32) for _ in range(5)]
    out = pl.pallas_call(
        _scan_kernel,
        grid=(BC // bcb,),
        in_specs=[spec] * 5,
        out_specs=spec,
        out_shape=jax.ShapeDtypeStruct((BC, H, W), jnp.float32),
        scratch_shapes=scratch,
        compiler_params=pltpu.CompilerParams(
            dimension_semantics=("parallel",),
            vmem_limit_bytes=100 * 1024 * 1024,
        ),
    )(*ins)
    return out.reshape(Bsz, C, H, W)
